# baseline (speedup 1.0000x reference)
import functools

import jax
import jax.numpy as jnp
from jax import lax
from jax.experimental import pallas as pl
from jax.experimental.pallas import tpu as pltpu
from jax.experimental.pallas import tpu_sc as plsc


_NC, _NS = 2, 16
_NW = _NC * _NS
_CHUNK = 16


@functools.lru_cache(maxsize=None)
def _make_sc_gather(N, V, H):
    rows_per_w = N // _NW
    n_ch = rows_per_w // _CHUNK
    mesh = plsc.VectorSubcoreMesh(core_axis_name="c", subcore_axis_name="s")

    @functools.partial(
        pl.kernel,
        mesh=mesh,
        out_type=jax.ShapeDtypeStruct((N, H), jnp.float32),
        scratch_types=[
            pltpu.VMEM((rows_per_w,), jnp.int32),
            pltpu.VMEM((_CHUNK, H), jnp.float32),
            pltpu.VMEM((_CHUNK, H), jnp.float32),
            pltpu.SemaphoreType.DMA,
            pltpu.SemaphoreType.DMA,
            pltpu.SemaphoreType.DMA,
            pltpu.SemaphoreType.DMA,
        ],
    )
    def sc_gather(idx_hbm, table_hbm, out_hbm, idx_v, rows0, rows1,
                  ga, gb, sa, sb):
        wid = lax.axis_index("s") * _NC + lax.axis_index("c")
        base = wid * rows_per_w
        pltpu.sync_copy(idx_hbm.at[pl.ds(base, rows_per_w)], idx_v)

        bufs = (rows0, rows1)
        gsems = (ga, gb)
        ssems = (sa, sb)

        def gather(c):
            return pltpu.async_copy(
                table_hbm.at[idx_v.at[pl.ds(c * _CHUNK, _CHUNK)]],
                bufs[c & 1], gsems[c & 1])

        def put(c):
            return pltpu.async_copy(
                bufs[c & 1],
                out_hbm.at[pl.ds(base + c * _CHUNK, _CHUNK)],
                ssems[c & 1])

        g = [None] * n_ch
        st = [None] * n_ch
        g[0] = gather(0)
        for c in range(n_ch):
            if c + 1 < n_ch:
                if c >= 1:
                    st[c - 1].wait()
                g[c + 1] = gather(c + 1)
            g[c].wait()
            st[c] = put(c)
        if n_ch >= 2:
            st[n_ch - 2].wait()
        st[n_ch - 1].wait()

    return sc_gather


@functools.lru_cache(maxsize=None)
def _make_tc_dense(S, B, H, BR):
    G = S // BR
    WS = S * B * 2
    WP = B * H
    WM = S * B

    def body(m_ref, q_ref, am_ref, pos_ref, if_ref,
             seg_ref, posout_ref, mask_ref):
        i = pl.program_id(0)

        lane = lax.broadcasted_iota(jnp.int32, (BR, WS), 1)
        bvec = (lane >> 1) & (B - 1)
        segi = (m_ref[...] >> bvec) & 1
        seg_ref[...] = (1 ^ segi ^ q_ref[...]).astype(jnp.float32)

        lane_m = lax.broadcasted_iota(jnp.int32, (BR, WM), 1)
        jvec = lane_m >> 2
        ivec = lax.broadcasted_iota(jnp.int32, (BR, WM), 0) + i * BR
        eye = (jvec == ivec).astype(jnp.float32)
        mask_ref[...] = ((am_ref[...] - eye) > 0).astype(jnp.float32)

        arg = pos_ref[...] * if_ref[...]
        half = H // 2
        val = jnp.concatenate(
            [jnp.sin(arg[:, :half]), jnp.cos(arg[:, half:])], axis=1)
        for r in range(B):
            posout_ref[:, r * H:(r + 1) * H] = val

    return pl.pallas_call(
        body,
        grid=(G,),
        in_specs=[
            pl.BlockSpec((BR, 1), lambda i: (i, 0)),
            pl.BlockSpec((1, WS), lambda i: (0, 0)),
            pl.BlockSpec((1, WM), lambda i: (0, 0)),
            pl.BlockSpec((2 * BR, 1), lambda i: (i, 0)),
            pl.BlockSpec((1, H), lambda i: (0, 0)),
        ],
        out_specs=[
            pl.BlockSpec((BR, WS), lambda i: (i, 0)),
            pl.BlockSpec((2 * BR, WP), lambda i: (i, 0)),
            pl.BlockSpec((BR, WM), lambda i: (i, 0)),
        ],
        out_shape=[
            jax.ShapeDtypeStruct((S, WS), jnp.float32),
            jax.ShapeDtypeStruct((2 * S, WP), jnp.float32),
            jax.ShapeDtypeStruct((S, WM), jnp.float32),
        ],
    )


def kernel(token_ids, segment_ids, attn_mask, token_embeddings):
    B, S = token_ids.shape
    V, H = token_embeddings.shape

    tid_t = token_ids.T
    seg_t = segment_ids.T
    am_t = attn_mask.T

    idx = tid_t.reshape(-1)
    weights = (1 << jnp.arange(B, dtype=jnp.int32))[None, :]
    m_col = jnp.sum(seg_t * weights, axis=1, dtype=jnp.int32)[:, None]
    q_row = (seg_t[:, :, None] ^ jnp.arange(2, dtype=jnp.int32)).reshape(1, -1)
    am_row = am_t.reshape(1, -1)
    pos_col = jnp.arange(S, -S, -1.0, dtype=jnp.float32)[:, None]
    freq_seq = jnp.arange(0, H, 2.0, dtype=jnp.float32)
    inv_freq = 1.0 / jnp.power(10000.0, freq_seq / H)
    if_row = jnp.concatenate([inv_freq, inv_freq]).reshape(1, H)

    token_embed_flat = _make_sc_gather(S * B, V, H)(idx, token_embeddings)
    seg_flat, pos_flat, mask_flat = _make_tc_dense(S, B, H, 32)(
        m_col, q_row, am_row, pos_col, if_row)

    return (
        token_embed_flat.reshape(S, B, H),
        seg_flat.reshape(S, S, B, 2),
        pos_flat.reshape(2 * S, B, H),
        mask_flat.reshape(S, S, B, 1),
    )


# speedup vs baseline: 1.9118x; 1.9118x over previous
import functools

import jax
import jax.numpy as jnp
from jax import lax
from jax.experimental import pallas as pl
from jax.experimental.pallas import tpu as pltpu
from jax.experimental.pallas import tpu_sc as plsc


_NC, _NS = 2, 16
_NW = _NC * _NS
_CHUNK = 16


@functools.lru_cache(maxsize=None)
def _make_sc_gather(S, B, V, H):
    HT = H // 128
    sg_per_b = _NW // B
    s_per_w = S // sg_per_b
    n_ch = s_per_w // _CHUNK
    mesh = plsc.VectorSubcoreMesh(core_axis_name="c", subcore_axis_name="s")

    @functools.partial(
        pl.kernel,
        mesh=mesh,
        out_type=jax.ShapeDtypeStruct((S, HT, B, 128), jnp.float32),
        scratch_types=[
            pltpu.VMEM((s_per_w,), jnp.int32),
            pltpu.VMEM((_CHUNK, H), jnp.float32),
            pltpu.VMEM((_CHUNK, H), jnp.float32),
            pltpu.SemaphoreType.DMA,
            pltpu.SemaphoreType.DMA,
            pltpu.SemaphoreType.DMA,
            pltpu.SemaphoreType.DMA,
        ],
    )
    def sc_gather(idx_hbm, table_hbm, out_hbm, idx_v, rows0, rows1,
                  ga, gb, sa, sb):
        wid = lax.axis_index("s") * _NC + lax.axis_index("c")
        b = wid // sg_per_b
        s0 = (wid % sg_per_b) * s_per_w
        pltpu.sync_copy(idx_hbm.at[pl.ds(wid * s_per_w, s_per_w)], idx_v)

        bufs = (rows0, rows1)
        gsems = (ga, gb)
        ssems = (sa, sb)

        def gather(c):
            return pltpu.async_copy(
                table_hbm.at[idx_v.at[pl.ds(c * _CHUNK, _CHUNK)]],
                bufs[c & 1], gsems[c & 1])

        def put(c):
            return [
                pltpu.async_copy(
                    bufs[c & 1].at[:, pl.ds(ht * 128, 128)],
                    out_hbm.at[pl.ds(s0 + c * _CHUNK, _CHUNK), ht, b, :],
                    ssems[c & 1])
                for ht in range(HT)
            ]

        def drain(copies):
            for cp in copies:
                cp.wait()

        g = [None] * n_ch
        st = [None] * n_ch
        g[0] = gather(0)
        for c in range(n_ch):
            if c + 1 < n_ch:
                if c >= 1:
                    drain(st[c - 1])
                g[c + 1] = gather(c + 1)
            g[c].wait()
            st[c] = put(c)
        if n_ch >= 2:
            drain(st[n_ch - 2])
        drain(st[n_ch - 1])

    return sc_gather


@functools.lru_cache(maxsize=None)
def _make_tc_dense(S, B, H, BR):
    G = S // BR
    ST = S // 128
    HT = H // 128
    GS = B * ST * 2
    GP = HT * B
    GM = B * ST

    def body(m_ref, q_ref, am_ref, pos_ref, if_ref,
             seg_ref, posout_ref, mask_ref):
        i = pl.program_id(0)

        gi = lax.broadcasted_iota(jnp.int32, (BR, GS, 128), 1)
        bvec = gi >> (2 * ST).bit_length() - 1
        segi = (m_ref[...][:, :, None] >> bvec) & 1
        seg_ref[...] = (1 ^ segi ^ q_ref[...]).astype(jnp.float32)

        gm = lax.broadcasted_iota(jnp.int32, (BR, GM, 128), 1)
        mm = lax.broadcasted_iota(jnp.int32, (BR, GM, 128), 2)
        jvec = ((gm & (ST - 1)) << 7) + mm
        ivec = lax.broadcasted_iota(jnp.int32, (BR, GM, 128), 0) + i * BR
        eye = (jvec == ivec).astype(jnp.float32)
        mask_ref[...] = ((am_ref[...] - eye) > 0).astype(jnp.float32)

        arg = pos_ref[...][:, :, None] * if_ref[...]
        half = GP // 2
        posout_ref[...] = jnp.concatenate(
            [jnp.sin(arg[:, :half, :]), jnp.cos(arg[:, half:, :])], axis=1)

    return pl.pallas_call(
        body,
        grid=(G,),
        in_specs=[
            pl.BlockSpec((BR, 1), lambda i: (i, 0)),
            pl.BlockSpec((1, GS, 128), lambda i: (0, 0, 0)),
            pl.BlockSpec((1, GM, 128), lambda i: (0, 0, 0)),
            pl.BlockSpec((2 * BR, 1), lambda i: (i, 0)),
            pl.BlockSpec((1, GP, 128), lambda i: (0, 0, 0)),
        ],
        out_specs=[
            pl.BlockSpec((BR, GS, 128), lambda i: (i, 0, 0)),
            pl.BlockSpec((2 * BR, GP, 128), lambda i: (i, 0, 0)),
            pl.BlockSpec((BR, GM, 128), lambda i: (i, 0, 0)),
        ],
        out_shape=[
            jax.ShapeDtypeStruct((S, GS, 128), jnp.float32),
            jax.ShapeDtypeStruct((2 * S, GP, 128), jnp.float32),
            jax.ShapeDtypeStruct((S, GM, 128), jnp.float32),
        ],
    )


def kernel(token_ids, segment_ids, attn_mask, token_embeddings):
    B, S = token_ids.shape
    V, H = token_embeddings.shape
    ST = S // 128
    HT = H // 128

    idx = token_ids.reshape(-1)
    seg32 = segment_ids.astype(jnp.int32)
    m_col = jnp.sum(seg32 * (1 << jnp.arange(B, dtype=jnp.int32))[:, None],
                    axis=0)[:, None]
    q3 = (seg32.reshape(B, ST, 1, 128)
          ^ jnp.arange(2, dtype=jnp.int32)[None, None, :, None]
          ).reshape(1, B * ST * 2, 128)
    am3 = attn_mask.reshape(1, B * ST, 128)
    pos_col = jnp.arange(S, -S, -1.0, dtype=jnp.float32)[:, None]
    freq_seq = jnp.arange(0, H, 2.0, dtype=jnp.float32)
    inv_freq = 1.0 / jnp.power(10000.0, freq_seq / H)
    ifd = jnp.concatenate([inv_freq, inv_freq])
    if3 = jnp.broadcast_to(ifd.reshape(HT, 1, 128),
                           (HT, B, 128)).reshape(1, HT * B, 128)

    tok4 = _make_sc_gather(S, B, V, H)(idx, token_embeddings)
    seg3, pos3, mask3 = _make_tc_dense(S, B, H, 32)(
        m_col, q3, am3, pos_col, if3)

    token_embed = tok4.transpose(0, 2, 1, 3).reshape(S, B, H)
    segment_embed = (seg3.reshape(S, B, ST, 2, 128)
                     .transpose(0, 2, 4, 1, 3).reshape(S, S, B, 2))
    pos_embed = (pos3.reshape(2 * S, HT, B, 128)
                 .transpose(0, 2, 1, 3).reshape(2 * S, B, H))
    non_target_mask = (mask3.reshape(S, B, ST, 128)
                       .transpose(0, 2, 3, 1).reshape(S, S, B, 1))
    return (token_embed, segment_embed, pos_embed, non_target_mask)
